# Initial kernel scaffold; baseline (speedup 1.0000x reference)
import jax
import jax.numpy as jnp
from jax.experimental import pallas as pl


def kernel(x, position_weight, value_weight):
    raise NotImplementedError("write your pallas kernel here")


# speedup vs baseline: 1.5648x; 1.5648x over previous
import functools

import jax
import jax.numpy as jnp
from jax import lax
from jax.experimental import pallas as pl
from jax.experimental.pallas import tpu as pltpu
from jax.experimental.pallas import tpu_sc as plsc

B = 64
N = 1024
D = 2048
L = 256
NC = 2
NS = 16
NW = NC * NS
DW = D // NW

_mesh = plsc.VectorSubcoreMesh(
    core_axis_name="c", subcore_axis_name="s", num_cores=NC, num_subcores=NS
)


@functools.partial(
    pl.kernel,
    out_type=jax.ShapeDtypeStruct((NW, B, 2, DW // 2), jnp.int16),
    mesh=_mesh,
    compiler_params=pltpu.CompilerParams(use_tc_tiling_on_sc=False),
    scratch_types=[
        pltpu.VMEM((N,), jnp.float32),
        pltpu.VMEM((N,), jnp.int32),
        pltpu.VMEM((N, 2, DW // 2), jnp.int16),
        pltpu.VMEM((L, 2, DW // 2), jnp.int16),
        pltpu.VMEM((B, 2, DW // 2), jnp.int16),
    ],
)
def _sc_encode(x_hbm, pos_hbm, val_hbm, out_hbm, xrow_v, idx_v, pos_v, val_v, out_v):
    c = lax.axis_index("c")
    s = lax.axis_index("s")
    wid = s * NC + c

    pltpu.sync_copy(pos_hbm.at[wid], pos_v)
    pltpu.sync_copy(val_hbm.at[wid], val_v)

    def acc_batch(b, carry):
        pltpu.sync_copy(x_hbm.at[b], xrow_v)

        def qg(g, carry2):
            v = xrow_v[pl.ds(g * 16, 16)] * 255.0
            t = v.astype(jnp.int32)
            f = v - t.astype(jnp.float32)
            up = (f > 0.5) | ((f == 0.5) & ((t & 1) == 1))
            r = jnp.where(up, t + 1, t)
            r = jnp.minimum(jnp.maximum(r, 0), 255)
            idx_v[pl.ds(g * 16, 16)] = r
            return carry2

        lax.fori_loop(0, N // 16, qg, 0)

        def gbody(g, accs):
            a0, a1 = accs
            iv = idx_v[pl.ds(g * 16, 16)]
            base = g * 16
            for j in range(16):
                n = base + j
                r = iv[j]
                a0 = a0 + pos_v[n, 0, pl.ds(0, 32)] * val_v[r, 0, pl.ds(0, 32)]
                a1 = a1 + pos_v[n, 1, pl.ds(0, 32)] * val_v[r, 1, pl.ds(0, 32)]
            return (a0, a1)

        z = jnp.zeros((32,), jnp.int16)
        a0, a1 = lax.fori_loop(0, N // 16, gbody, (z, z))
        one = jnp.int16(1)
        mone = jnp.int16(-1)
        s0 = jnp.maximum(jnp.minimum(a0 - one, one), mone)
        s1 = jnp.maximum(jnp.minimum(a1 - one, one), mone)
        out_v[b, 0, pl.ds(0, 32)] = s0
        out_v[b, 1, pl.ds(0, 32)] = s1
        return carry

    lax.fori_loop(0, B, acc_batch, 0)
    pltpu.sync_copy(out_v, out_hbm.at[wid])


def kernel(x, position_weight, value_weight):
    xf = x.reshape(B, N)
    pos16 = (
        position_weight.astype(jnp.int16)
        .reshape(N, NW, 2, DW // 2)
        .transpose(1, 0, 2, 3)
    )
    val16 = (
        value_weight.astype(jnp.int16)
        .reshape(L, NW, 2, DW // 2)
        .transpose(1, 0, 2, 3)
    )
    out_t = _sc_encode(xf, pos16, val16)
    return (
        out_t.reshape(NW, B, DW).transpose(1, 0, 2).reshape(B, D).astype(jnp.float32)
    )


# speedup vs baseline: 1.8206x; 1.1635x over previous
import functools

import jax
import jax.numpy as jnp
from jax import lax
from jax.experimental import pallas as pl
from jax.experimental.pallas import tpu as pltpu
from jax.experimental.pallas import tpu_sc as plsc

B = 64
N = 1024
D = 2048
L = 256
NC = 2
NS = 16
NW = NC * NS
DW = D // NW
G = 4

_mesh = plsc.VectorSubcoreMesh(
    core_axis_name="c", subcore_axis_name="s", num_cores=NC, num_subcores=NS
)


@functools.partial(
    pl.kernel,
    out_type=jax.ShapeDtypeStruct((NW, B, 2, DW // 2), jnp.int16),
    mesh=_mesh,
    compiler_params=pltpu.CompilerParams(use_tc_tiling_on_sc=False),
    scratch_types=[
        pltpu.VMEM((G, N), jnp.float32),
        pltpu.VMEM((G, N), jnp.int32),
        pltpu.VMEM((N, 2, DW // 2), jnp.int16),
        pltpu.VMEM((L, 2, DW // 2), jnp.int16),
        pltpu.VMEM((B, 2, DW // 2), jnp.int16),
    ],
)
def _sc_encode(x_hbm, pos_hbm, val_hbm, out_hbm, xrow_v, idx_v, pos_v, val_v, out_v):
    c = lax.axis_index("c")
    s = lax.axis_index("s")
    wid = s * NC + c

    pltpu.sync_copy(pos_hbm.at[wid], pos_v)
    pltpu.sync_copy(val_hbm.at[wid], val_v)

    def group_body(bg, carry):
        pltpu.sync_copy(x_hbm.at[pl.ds(bg * G, G)], xrow_v)

        def qg(g, carry2):
            for k in range(G):
                v = xrow_v[k, pl.ds(g * 16, 16)] * 255.0
                t = v.astype(jnp.int32)
                f = v - t.astype(jnp.float32)
                up = (f > 0.5) | ((f == 0.5) & ((t & 1) == 1))
                r = jnp.where(up, t + 1, t)
                r = jnp.minimum(jnp.maximum(r, 0), 255)
                idx_v[k, pl.ds(g * 16, 16)] = r
            return carry2

        lax.fori_loop(0, N // 16, qg, 0)

        def gbody(g, accs):
            accs = list(accs)
            ivs = [idx_v[k, pl.ds(g * 16, 16)] for k in range(G)]
            base = g * 16
            for j in range(16):
                n = base + j
                p0 = pos_v[n, 0, pl.ds(0, 32)]
                p1 = pos_v[n, 1, pl.ds(0, 32)]
                for k in range(G):
                    r = ivs[k][j]
                    accs[2 * k] = accs[2 * k] + p0 * val_v[r, 0, pl.ds(0, 32)]
                    accs[2 * k + 1] = accs[2 * k + 1] + p1 * val_v[r, 1, pl.ds(0, 32)]
            return tuple(accs)

        z = jnp.zeros((32,), jnp.int16)
        accs = lax.fori_loop(0, N // 16, gbody, (z,) * (2 * G))
        one = jnp.int16(1)
        mone = jnp.int16(-1)
        for k in range(G):
            s0 = jnp.maximum(jnp.minimum(accs[2 * k] - one, one), mone)
            s1 = jnp.maximum(jnp.minimum(accs[2 * k + 1] - one, one), mone)
            out_v[bg * G + k, 0, pl.ds(0, 32)] = s0
            out_v[bg * G + k, 1, pl.ds(0, 32)] = s1
        return carry

    lax.fori_loop(0, B // G, group_body, 0)
    pltpu.sync_copy(out_v, out_hbm.at[wid])


def kernel(x, position_weight, value_weight):
    xf = x.reshape(B, N)
    pos16 = (
        position_weight.astype(jnp.int16)
        .reshape(N, NW, 2, DW // 2)
        .transpose(1, 0, 2, 3)
    )
    val16 = (
        value_weight.astype(jnp.int16)
        .reshape(L, NW, 2, DW // 2)
        .transpose(1, 0, 2, 3)
    )
    out_t = _sc_encode(xf, pos16, val16)
    return (
        out_t.reshape(NW, B, DW).transpose(1, 0, 2).reshape(B, D).astype(jnp.float32)
    )
